# Initial kernel scaffold; baseline (speedup 1.0000x reference)
import functools

import jax
import jax.numpy as jnp
import numpy as np
from jax import lax
from jax.experimental import pallas as pl
from jax.experimental.pallas import tpu as pltpu
from jax.experimental.pallas import tpu_sc as plsc

N = 10000
NPAD = 10240
HID = 128
H = 8
DH = 16
E = 320000
ES = 100000
ESP = 100096

NC = 2
NS = 16
NW = NC * NS
K = 128

_BLK = 1024


def _mm_relu_body(x_ref, w_ref, b_ref, o_ref):
    o_ref[...] = jax.nn.relu(
        jnp.dot(x_ref[...], w_ref[...], preferred_element_type=jnp.float32)
        + b_ref[...])


def _mm_relu(x, w, b):
    return pl.pallas_call(
        _mm_relu_body,
        grid=(NPAD // _BLK,),
        in_specs=[pl.BlockSpec((_BLK, HID), lambda i: (i, 0)),
                  pl.BlockSpec((HID, HID), lambda i: (0, 0)),
                  pl.BlockSpec((1, HID), lambda i: (0, 0))],
        out_specs=pl.BlockSpec((_BLK, HID), lambda i: (i, 0)),
        out_shape=jax.ShapeDtypeStruct((NPAD, HID), jnp.float32),
    )(x, w, b.reshape(1, HID))


def _proj_body(x_ref, w_ref, b_ref, am_ref, p_ref, a_ref):
    p = (jnp.dot(x_ref[...], w_ref[...], preferred_element_type=jnp.float32)
         + b_ref[...])
    p_ref[...] = p
    a_ref[...] = jnp.dot(p, am_ref[...], preferred_element_type=jnp.float32)


def _proj(x, w, b, am):
    return pl.pallas_call(
        _proj_body,
        grid=(NPAD // _BLK,),
        in_specs=[pl.BlockSpec((_BLK, HID), lambda i: (i, 0)),
                  pl.BlockSpec((HID, HID), lambda i: (0, 0)),
                  pl.BlockSpec((1, HID), lambda i: (0, 0)),
                  pl.BlockSpec((HID, 32), lambda i: (0, 0))],
        out_specs=[pl.BlockSpec((_BLK, HID), lambda i: (i, 0)),
                   pl.BlockSpec((_BLK, 32), lambda i: (i, 0))],
        out_shape=[jax.ShapeDtypeStruct((NPAD, HID), jnp.float32),
                   jax.ShapeDtypeStruct((NPAD, 32), jnp.float32)],
    )(x, w, b.reshape(1, HID), am)


_R_EXPAND = np.zeros((16, HID), np.float32)
for _h in range(H):
    _R_EXPAND[_h, _h * DH:(_h + 1) * DH] = 1.0


def _norm_body(msg_ref, den_ref, r_ref, o_ref):
    m = msg_ref[0] + msg_ref[1]
    d = den_ref[0] + den_ref[1]
    db = jnp.dot(d, r_ref[...], preferred_element_type=jnp.float32)
    o_ref[...] = jax.nn.relu(m / (db + 1e-16))


def _norm(msg, den):
    return pl.pallas_call(
        _norm_body,
        grid=(NPAD // _BLK,),
        in_specs=[pl.BlockSpec((NC, _BLK, HID), lambda i: (0, i, 0)),
                  pl.BlockSpec((NC, _BLK, 16), lambda i: (0, i, 0)),
                  pl.BlockSpec((16, HID), lambda i: (0, 0))],
        out_specs=pl.BlockSpec((_BLK, HID), lambda i: (i, 0)),
        out_shape=jax.ShapeDtypeStruct((NPAD, HID), jnp.float32),
    )(msg, den, jnp.asarray(_R_EXPAND))


def _edge_pass(a_s, a_d, p_src, j_idx, i_idx, zmsg, zden):
    n_chunks = E // K
    base_ch = n_chunks // NW
    rem_ch = n_chunks % NW
    rows_t = NPAD // NS
    mesh = plsc.VectorSubcoreMesh(core_axis_name="c", subcore_axis_name="s")

    @functools.partial(
        pl.kernel,
        out_type=(jax.ShapeDtypeStruct((NC, NPAD, HID), jnp.float32),
                  jax.ShapeDtypeStruct((NC, NPAD, 16), jnp.float32)),
        mesh=mesh,
        scratch_types=[
            pltpu.VMEM((K,), jnp.int32),
            pltpu.VMEM((K,), jnp.int32),
            pltpu.VMEM((K, 16), jnp.float32),
            pltpu.VMEM((K, 16), jnp.float32),
            pltpu.VMEM((K, HID), jnp.float32),
            pltpu.VMEM((K, HID), jnp.float32),
            pltpu.VMEM((K, 16), jnp.float32),
            pltpu.VMEM_SHARED((NPAD, HID), jnp.float32),
            pltpu.VMEM_SHARED((NPAD, 16), jnp.float32),
            pltpu.SemaphoreType.DMA,
            pltpu.SemaphoreType.DMA,
            pltpu.SemaphoreType.DMA,
        ],
    )
    def k(a_s_hbm, a_d_hbm, p_hbm, j_hbm, i_hbm, zmsg_hbm, zden_hbm,
          msg_out, den_out,
          idx_j, idx_i, as_rows, ad_rows, xs_rows, msg_buf, ex_buf,
          msg_acc, den_acc, sem0, sem1, sem2):
        c = lax.axis_index("c")
        s = lax.axis_index("s")
        wid = s * NC + c
        r0 = s * rows_t
        pltpu.sync_copy(zmsg_hbm.at[pl.ds(r0, rows_t)],
                        msg_acc.at[pl.ds(r0, rows_t)])
        pltpu.sync_copy(zden_hbm.at[pl.ds(r0, rows_t)],
                        den_acc.at[pl.ds(r0, rows_t)])
        plsc.subcore_barrier()

        nt = base_ch + jnp.where(wid < rem_ch, 1, 0)

        def chunk_body(t, carry):
            ch = wid + NW * t
            base = ch * K
            pltpu.sync_copy(j_hbm.at[pl.ds(base, K)], idx_j)
            pltpu.sync_copy(i_hbm.at[pl.ds(base, K)], idx_i)
            cp0 = pltpu.async_copy(a_s_hbm.at[idx_j], as_rows, sem0)
            cp1 = pltpu.async_copy(a_d_hbm.at[idx_i], ad_rows, sem1)
            cp2 = pltpu.async_copy(p_hbm.at[idx_j], xs_rows, sem2)
            cp0.wait()
            cp1.wait()
            cp2.wait()

            def edge_body(e, inner):
                al = as_rows[e] + ad_rows[e]
                al = jnp.where(al >= 0.0, al, al * 0.2)
                ex_buf[e] = jnp.exp(al)
                for h in range(H):
                    exh = ex_buf[e, h]
                    sl = pl.ds(h * DH, DH)
                    msg_buf[e, sl] = exh * xs_rows[e, sl]
                return inner

            lax.fori_loop(0, K, edge_body, 0)
            pltpu.sync_copy(msg_buf, msg_acc.at[idx_i], add=True)
            pltpu.sync_copy(ex_buf, den_acc.at[idx_i], add=True)
            return carry

        lax.fori_loop(0, nt, chunk_body, 0)
        plsc.subcore_barrier()
        pltpu.sync_copy(msg_acc.at[pl.ds(r0, rows_t)],
                        msg_out.at[c, pl.ds(r0, rows_t)])
        pltpu.sync_copy(den_acc.at[pl.ds(r0, rows_t)],
                        den_out.at[c, pl.ds(r0, rows_t)])

    return k(a_s, a_d, p_src, j_idx, i_idx, zmsg, zden)


def _pair_dot(t1a, t1b, t2a, t2b, m_idx, d_idx):
    n_chunks = ESP // K
    base_ch = n_chunks // NW
    rem_ch = n_chunks % NW
    mesh = plsc.VectorSubcoreMesh(core_axis_name="c", subcore_axis_name="s")

    @functools.partial(
        pl.kernel,
        out_type=jax.ShapeDtypeStruct((ESP,), jnp.float32),
        mesh=mesh,
        scratch_types=[
            pltpu.VMEM((K,), jnp.int32),
            pltpu.VMEM((K,), jnp.int32),
            pltpu.VMEM((K, HID), jnp.float32),
            pltpu.VMEM((K, HID), jnp.float32),
            pltpu.VMEM((K, HID), jnp.float32),
            pltpu.VMEM((K, HID), jnp.float32),
            pltpu.VMEM((K,), jnp.float32),
            pltpu.SemaphoreType.DMA,
            pltpu.SemaphoreType.DMA,
            pltpu.SemaphoreType.DMA,
            pltpu.SemaphoreType.DMA,
        ],
    )
    def k(t1a_hbm, t1b_hbm, t2a_hbm, t2b_hbm, m_hbm, d_hbm, y_hbm,
          mi, di, r1a, r1b, r2a, r2b, ybuf, sa, sb, sc2, sd):
        c = lax.axis_index("c")
        s = lax.axis_index("s")
        wid = s * NC + c
        nt = base_ch + jnp.where(wid < rem_ch, 1, 0)

        def chunk_body(t, carry):
            ch = wid + NW * t
            base = ch * K
            pltpu.sync_copy(m_hbm.at[pl.ds(base, K)], mi)
            pltpu.sync_copy(d_hbm.at[pl.ds(base, K)], di)
            cpa = pltpu.async_copy(t1a_hbm.at[mi], r1a, sa)
            cpb = pltpu.async_copy(t1b_hbm.at[mi], r1b, sb)
            cpc = pltpu.async_copy(t2a_hbm.at[di], r2a, sc2)
            cpd = pltpu.async_copy(t2b_hbm.at[di], r2b, sd)
            cpa.wait()
            cpb.wait()
            cpc.wait()
            cpd.wait()

            def pair_body(e, inner):
                acc = r1a[e, pl.ds(0, 16)] * r2a[e, pl.ds(0, 16)]
                acc = acc + r1b[e, pl.ds(0, 16)] * r2b[e, pl.ds(0, 16)]
                for hh in range(1, HID // 16):
                    sl = pl.ds(hh * 16, 16)
                    acc = acc + r1a[e, sl] * r2a[e, sl]
                    acc = acc + r1b[e, sl] * r2b[e, sl]
                ybuf[e] = jnp.sum(acc)
                return inner

            lax.fori_loop(0, K, pair_body, 0)
            pltpu.sync_copy(ybuf, y_hbm.at[pl.ds(base, K)])
            return carry

        lax.fori_loop(0, nt, chunk_body, 0)

    return k(t1a, t1b, t2a, t2b, m_idx, d_idx)


def _amat(att):
    eye = jnp.eye(H, dtype=jnp.float32)
    return (att[:, :, None] * eye[:, None, :]).reshape(HID, H)


def kernel(x_n1, x_n2, edge_index_n12, edge_index_n21, edge_index, params):
    f32 = jnp.float32
    x1 = jnp.pad(x_n1.astype(f32), ((0, NPAD - N), (0, 0)))
    x2 = jnp.pad(x_n2.astype(f32), ((0, NPAD - N), (0, 0)))
    j12 = edge_index_n12[0].astype(jnp.int32)
    i12 = edge_index_n12[1].astype(jnp.int32)
    j21 = edge_index_n21[0].astype(jnp.int32)
    i21 = edge_index_n21[1].astype(jnp.int32)
    mi = jnp.pad(edge_index[0].astype(jnp.int32), (0, ESP - ES))
    di = jnp.pad(edge_index[1].astype(jnp.int32), (0, ESP - ES))
    zmsg = jnp.zeros((NPAD, HID), f32)
    zden = jnp.zeros((NPAD, 16), f32)

    h1 = _mm_relu(x1, params['lin']['n1']['W'], params['lin']['n1']['b'])
    h2 = _mm_relu(x2, params['lin']['n2']['W'], params['lin']['n2']['b'])

    outs = []
    for lp in params['layers']:
        am1 = jnp.zeros((HID, 32), f32)
        am1 = am1.at[:, 0:H].set(_amat(lp['att']['n1->n2']['src']))
        am1 = am1.at[:, 16:16 + H].set(_amat(lp['att']['n2->n1']['dst']))
        am2 = jnp.zeros((HID, 32), f32)
        am2 = am2.at[:, 0:H].set(_amat(lp['att']['n2->n1']['src']))
        am2 = am2.at[:, 16:16 + H].set(_amat(lp['att']['n1->n2']['dst']))

        p1, a1 = _proj(h1, lp['proj']['n1']['W'], lp['proj']['n1']['b'], am1)
        p2, a2 = _proj(h2, lp['proj']['n2']['W'], lp['proj']['n2']['b'], am2)
        a1s, a1d = a1[:, :16], a1[:, 16:]
        a2s, a2d = a2[:, :16], a2[:, 16:]

        msg2, den2 = _edge_pass(a1s, a2d, p1, j12, i12, zmsg, zden)
        msg1, den1 = _edge_pass(a2s, a1d, p2, j21, i21, zmsg, zden)
        h1 = _norm(msg1, den1)
        h2 = _norm(msg2, den2)
        outs.append((h1, h2))

    y = _pair_dot(outs[0][0], outs[1][0], outs[0][1], outs[1][1], mi, di)
    return y[:ES].reshape(ES, 1)


# speedup vs baseline: 46.7614x; 46.7614x over previous
import functools

import jax
import jax.numpy as jnp
import numpy as np
from jax import lax
from jax.experimental import pallas as pl
from jax.experimental.pallas import tpu as pltpu
from jax.experimental.pallas import tpu_sc as plsc

N = 10000
NPAD = 10112
HID = 128
H = 8
DH = 16
E = 320000
ES = 100000
ESP = 100096

NC = 2
NS = 16
NW = NC * NS
K = 128

_BLK = 632


def _mm_relu_body(x_ref, w_ref, b_ref, o_ref):
    o_ref[...] = jax.nn.relu(
        jnp.dot(x_ref[...], w_ref[...], preferred_element_type=jnp.float32)
        + b_ref[...])


def _mm_relu(x, w, b):
    return pl.pallas_call(
        _mm_relu_body,
        grid=(NPAD // _BLK,),
        in_specs=[pl.BlockSpec((_BLK, HID), lambda i: (i, 0)),
                  pl.BlockSpec((HID, HID), lambda i: (0, 0)),
                  pl.BlockSpec((1, HID), lambda i: (0, 0))],
        out_specs=pl.BlockSpec((_BLK, HID), lambda i: (i, 0)),
        out_shape=jax.ShapeDtypeStruct((NPAD, HID), jnp.float32),
    )(x, w, b.reshape(1, HID))


def _proj_body(x_ref, w_ref, b_ref, am_ref, p_ref, a_ref):
    p = (jnp.dot(x_ref[...], w_ref[...], preferred_element_type=jnp.float32)
         + b_ref[...])
    p_ref[...] = p
    a_ref[...] = jnp.dot(p, am_ref[...], preferred_element_type=jnp.float32)


def _proj(x, w, b, am):
    return pl.pallas_call(
        _proj_body,
        grid=(NPAD // _BLK,),
        in_specs=[pl.BlockSpec((_BLK, HID), lambda i: (i, 0)),
                  pl.BlockSpec((HID, HID), lambda i: (0, 0)),
                  pl.BlockSpec((1, HID), lambda i: (0, 0)),
                  pl.BlockSpec((HID, 32), lambda i: (0, 0))],
        out_specs=[pl.BlockSpec((_BLK, HID), lambda i: (i, 0)),
                   pl.BlockSpec((_BLK, 32), lambda i: (i, 0))],
        out_shape=[jax.ShapeDtypeStruct((NPAD, HID), jnp.float32),
                   jax.ShapeDtypeStruct((NPAD, 32), jnp.float32)],
    )(x, w, b.reshape(1, HID), am)


_R_EXPAND = np.zeros((16, HID), np.float32)
for _h in range(H):
    _R_EXPAND[_h, _h * DH:(_h + 1) * DH] = 1.0


def _norm_body(msg_ref, den_ref, r_ref, o_ref):
    m = msg_ref[0] + msg_ref[1]
    d = den_ref[0] + den_ref[1]
    db = jnp.dot(d, r_ref[...], preferred_element_type=jnp.float32)
    o_ref[...] = jax.nn.relu(m / (db + 1e-16))


def _norm(msg, den):
    return pl.pallas_call(
        _norm_body,
        grid=(NPAD // _BLK,),
        in_specs=[pl.BlockSpec((NC, _BLK, HID), lambda i: (0, i, 0)),
                  pl.BlockSpec((NC, _BLK, 16), lambda i: (0, i, 0)),
                  pl.BlockSpec((16, HID), lambda i: (0, 0))],
        out_specs=pl.BlockSpec((_BLK, HID), lambda i: (i, 0)),
        out_shape=jax.ShapeDtypeStruct((NPAD, HID), jnp.float32),
    )(msg, den, jnp.asarray(_R_EXPAND))


def _edge_pass(a_s, a_d, p_src, j_idx, i_idx, zmsg, zden):
    n_chunks = E // K
    base_ch = n_chunks // NW
    rem_ch = n_chunks % NW
    rows_t = NPAD // NS
    mesh = plsc.VectorSubcoreMesh(core_axis_name="c", subcore_axis_name="s")

    @functools.partial(
        pl.kernel,
        out_type=(jax.ShapeDtypeStruct((NC, NPAD, HID), jnp.float32),
                  jax.ShapeDtypeStruct((NC, NPAD, 16), jnp.float32)),
        mesh=mesh,
        compiler_params=pltpu.CompilerParams(use_tc_tiling_on_sc=False),
        scratch_types=[
            pltpu.VMEM((K,), jnp.int32),
            pltpu.VMEM((K,), jnp.int32),
            pltpu.VMEM((K, 16), jnp.float32),
            pltpu.VMEM((K, 16), jnp.float32),
            pltpu.VMEM((K, HID), jnp.float32),
            pltpu.VMEM((K, HID), jnp.float32),
            pltpu.VMEM((K, 16), jnp.float32),
            pltpu.VMEM_SHARED((NPAD, HID), jnp.float32),
            pltpu.VMEM_SHARED((NPAD, 16), jnp.float32),
            pltpu.SemaphoreType.DMA,
            pltpu.SemaphoreType.DMA,
            pltpu.SemaphoreType.DMA,
        ],
    )
    def k(a_s_hbm, a_d_hbm, p_hbm, j_hbm, i_hbm, zmsg_hbm, zden_hbm,
          msg_out, den_out,
          idx_j, idx_i, as_rows, ad_rows, xs_rows, msg_buf, ex_buf,
          msg_acc, den_acc, sem0, sem1, sem2):
        c = lax.axis_index("c")
        s = lax.axis_index("s")
        wid = s * NC + c
        r0 = s * rows_t
        pltpu.sync_copy(zmsg_hbm.at[pl.ds(r0, rows_t)],
                        msg_acc.at[pl.ds(r0, rows_t)])
        pltpu.sync_copy(zden_hbm.at[pl.ds(r0, rows_t)],
                        den_acc.at[pl.ds(r0, rows_t)])
        plsc.subcore_barrier()

        nt = base_ch + jnp.where(wid < rem_ch, 1, 0)

        def chunk_body(t, carry):
            ch = wid + NW * t
            base = ch * K
            pltpu.sync_copy(j_hbm.at[pl.ds(base, K)], idx_j)
            pltpu.sync_copy(i_hbm.at[pl.ds(base, K)], idx_i)
            cp0 = pltpu.async_copy(a_s_hbm.at[idx_j], as_rows, sem0)
            cp1 = pltpu.async_copy(a_d_hbm.at[idx_i], ad_rows, sem1)
            cp2 = pltpu.async_copy(p_hbm.at[idx_j], xs_rows, sem2)
            cp0.wait()
            cp1.wait()
            cp2.wait()

            def edge_body(e, inner):
                al = as_rows[e] + ad_rows[e]
                al = jnp.where(al >= 0.0, al, al * 0.2)
                ex = jnp.exp(al)
                ex_buf[e] = ex
                for h in range(H):
                    exh = ex[h]
                    sl = pl.ds(h * DH, DH)
                    msg_buf[e, sl] = exh * xs_rows[e, sl]
                return inner

            lax.fori_loop(0, K, edge_body, 0)
            pltpu.sync_copy(msg_buf, msg_acc.at[idx_i], add=True)
            pltpu.sync_copy(ex_buf, den_acc.at[idx_i], add=True)
            return carry

        lax.fori_loop(0, nt, chunk_body, 0)
        plsc.subcore_barrier()
        pltpu.sync_copy(msg_acc.at[pl.ds(r0, rows_t)],
                        msg_out.at[c, pl.ds(r0, rows_t)])
        pltpu.sync_copy(den_acc.at[pl.ds(r0, rows_t)],
                        den_out.at[c, pl.ds(r0, rows_t)])

    return k(a_s, a_d, p_src, j_idx, i_idx, zmsg, zden)


def _pair_dot(t1a, t1b, t2a, t2b, m_idx, d_idx):
    n_chunks = ESP // K
    base_ch = n_chunks // NW
    rem_ch = n_chunks % NW
    mesh = plsc.VectorSubcoreMesh(core_axis_name="c", subcore_axis_name="s")

    @functools.partial(
        pl.kernel,
        out_type=jax.ShapeDtypeStruct((ESP,), jnp.float32),
        mesh=mesh,
        compiler_params=pltpu.CompilerParams(use_tc_tiling_on_sc=False),
        scratch_types=[
            pltpu.VMEM((K,), jnp.int32),
            pltpu.VMEM((K,), jnp.int32),
            pltpu.VMEM((K, HID), jnp.float32),
            pltpu.VMEM((K, HID), jnp.float32),
            pltpu.VMEM((K, HID), jnp.float32),
            pltpu.VMEM((K, HID), jnp.float32),
            pltpu.VMEM((K,), jnp.float32),
            pltpu.SemaphoreType.DMA,
            pltpu.SemaphoreType.DMA,
            pltpu.SemaphoreType.DMA,
            pltpu.SemaphoreType.DMA,
        ],
    )
    def k(t1a_hbm, t1b_hbm, t2a_hbm, t2b_hbm, m_hbm, d_hbm, y_hbm,
          mi, di, r1a, r1b, r2a, r2b, ybuf, sa, sb, sc2, sd):
        c = lax.axis_index("c")
        s = lax.axis_index("s")
        wid = s * NC + c
        nt = base_ch + jnp.where(wid < rem_ch, 1, 0)

        def chunk_body(t, carry):
            ch = wid + NW * t
            base = ch * K
            pltpu.sync_copy(m_hbm.at[pl.ds(base, K)], mi)
            pltpu.sync_copy(d_hbm.at[pl.ds(base, K)], di)
            cpa = pltpu.async_copy(t1a_hbm.at[mi], r1a, sa)
            cpb = pltpu.async_copy(t1b_hbm.at[mi], r1b, sb)
            cpc = pltpu.async_copy(t2a_hbm.at[di], r2a, sc2)
            cpd = pltpu.async_copy(t2b_hbm.at[di], r2b, sd)
            cpa.wait()
            cpb.wait()
            cpc.wait()
            cpd.wait()

            lanes = lax.iota(jnp.int32, 16)

            def group_body(g, carry):
                def pair_body(p, y16):
                    e = g * 16 + p
                    acc = r1a[e, pl.ds(0, 16)] * r2a[e, pl.ds(0, 16)]
                    acc = acc + r1b[e, pl.ds(0, 16)] * r2b[e, pl.ds(0, 16)]
                    for hh in range(1, HID // 16):
                        sl = pl.ds(hh * 16, 16)
                        acc = acc + r1a[e, sl] * r2a[e, sl]
                        acc = acc + r1b[e, sl] * r2b[e, sl]
                    for sh in (8, 4, 2, 1):
                        acc = acc + acc.at[lanes ^ sh].get(
                            mode="promise_in_bounds")
                    return jnp.where(lanes == p, acc, y16)

                y16 = lax.fori_loop(0, 16, pair_body,
                                    jnp.zeros((16,), jnp.float32))
                ybuf[pl.ds(g * 16, 16)] = y16
                return carry

            lax.fori_loop(0, K // 16, group_body, 0)
            pltpu.sync_copy(ybuf, y_hbm.at[pl.ds(base, K)])
            return carry

        lax.fori_loop(0, nt, chunk_body, 0)

    return k(t1a, t1b, t2a, t2b, m_idx, d_idx)


def _amat(att):
    eye = jnp.eye(H, dtype=jnp.float32)
    return (att[:, :, None] * eye[:, None, :]).reshape(HID, H)


def kernel(x_n1, x_n2, edge_index_n12, edge_index_n21, edge_index, params):
    f32 = jnp.float32
    x1 = jnp.pad(x_n1.astype(f32), ((0, NPAD - N), (0, 0)))
    x2 = jnp.pad(x_n2.astype(f32), ((0, NPAD - N), (0, 0)))
    j12 = edge_index_n12[0].astype(jnp.int32)
    i12 = edge_index_n12[1].astype(jnp.int32)
    j21 = edge_index_n21[0].astype(jnp.int32)
    i21 = edge_index_n21[1].astype(jnp.int32)
    mi = jnp.pad(edge_index[0].astype(jnp.int32), (0, ESP - ES))
    di = jnp.pad(edge_index[1].astype(jnp.int32), (0, ESP - ES))
    zmsg = jnp.zeros((NPAD, HID), f32)
    zden = jnp.zeros((NPAD, 16), f32)

    h1 = _mm_relu(x1, params['lin']['n1']['W'], params['lin']['n1']['b'])
    h2 = _mm_relu(x2, params['lin']['n2']['W'], params['lin']['n2']['b'])

    outs = []
    for lp in params['layers']:
        am1 = jnp.zeros((HID, 32), f32)
        am1 = am1.at[:, 0:H].set(_amat(lp['att']['n1->n2']['src']))
        am1 = am1.at[:, 16:16 + H].set(_amat(lp['att']['n2->n1']['dst']))
        am2 = jnp.zeros((HID, 32), f32)
        am2 = am2.at[:, 0:H].set(_amat(lp['att']['n2->n1']['src']))
        am2 = am2.at[:, 16:16 + H].set(_amat(lp['att']['n1->n2']['dst']))

        p1, a1 = _proj(h1, lp['proj']['n1']['W'], lp['proj']['n1']['b'], am1)
        p2, a2 = _proj(h2, lp['proj']['n2']['W'], lp['proj']['n2']['b'], am2)
        a1s, a1d = a1[:, :16], a1[:, 16:]
        a2s, a2d = a2[:, :16], a2[:, 16:]

        msg2, den2 = _edge_pass(a1s, a2d, p1, j12, i12, zmsg, zden)
        msg1, den1 = _edge_pass(a2s, a1d, p2, j21, i21, zmsg, zden)
        h1 = _norm(msg1, den1)
        h2 = _norm(msg2, den2)
        outs.append((h1, h2))

    y = _pair_dot(outs[0][0], outs[1][0], outs[0][1], outs[1][1], mi, di)
    return y[:ES].reshape(ES, 1)


# speedup vs baseline: 48.4818x; 1.0368x over previous
import functools

import jax
import jax.numpy as jnp
import numpy as np
from jax import lax
from jax.experimental import pallas as pl
from jax.experimental.pallas import tpu as pltpu
from jax.experimental.pallas import tpu_sc as plsc

N = 10000
NPAD = 10112
HID = 128
H = 8
DH = 16
E = 320000
ES = 100000
ESP = 100096

NC = 2
NS = 16
NW = NC * NS
K = 128
WMSG = HID + 16

_BLK = 632


def _mm_relu_body(x_ref, w_ref, b_ref, o_ref):
    o_ref[...] = jax.nn.relu(
        jnp.dot(x_ref[...], w_ref[...], preferred_element_type=jnp.float32)
        + b_ref[...])


def _mm_relu(x, w, b):
    return pl.pallas_call(
        _mm_relu_body,
        grid=(NPAD // _BLK,),
        in_specs=[pl.BlockSpec((_BLK, HID), lambda i: (i, 0)),
                  pl.BlockSpec((HID, HID), lambda i: (0, 0)),
                  pl.BlockSpec((1, HID), lambda i: (0, 0))],
        out_specs=pl.BlockSpec((_BLK, HID), lambda i: (i, 0)),
        out_shape=jax.ShapeDtypeStruct((NPAD, HID), jnp.float32),
    )(x, w, b.reshape(1, HID))


def _proj_body(x_ref, w_ref, b_ref, ams_ref, amd_ref, pe_ref, ad_ref):
    p = (jnp.dot(x_ref[...], w_ref[...], preferred_element_type=jnp.float32)
         + b_ref[...])
    pe_ref[:, :HID] = p
    pe_ref[:, HID:WMSG] = jnp.dot(p, ams_ref[...],
                                  preferred_element_type=jnp.float32)
    ad_ref[...] = jnp.dot(p, amd_ref[...], preferred_element_type=jnp.float32)


def _proj(x, w, b, ams, amd):
    return pl.pallas_call(
        _proj_body,
        grid=(NPAD // _BLK,),
        in_specs=[pl.BlockSpec((_BLK, HID), lambda i: (i, 0)),
                  pl.BlockSpec((HID, HID), lambda i: (0, 0)),
                  pl.BlockSpec((1, HID), lambda i: (0, 0)),
                  pl.BlockSpec((HID, 16), lambda i: (0, 0)),
                  pl.BlockSpec((HID, 16), lambda i: (0, 0))],
        out_specs=[pl.BlockSpec((_BLK, WMSG), lambda i: (i, 0)),
                   pl.BlockSpec((_BLK, 16), lambda i: (i, 0))],
        out_shape=[jax.ShapeDtypeStruct((NPAD, WMSG), jnp.float32),
                   jax.ShapeDtypeStruct((NPAD, 16), jnp.float32)],
    )(x, w, b.reshape(1, HID), ams, amd)


_R_EXPAND = np.zeros((16, HID), np.float32)
for _h in range(H):
    _R_EXPAND[_h, _h * DH:(_h + 1) * DH] = 1.0


def _norm_body(acc_ref, r_ref, o_ref):
    a = acc_ref[0] + acc_ref[1]
    m = a[:, :HID]
    d = a[:, HID:WMSG]
    db = jnp.dot(d, r_ref[...], preferred_element_type=jnp.float32)
    o_ref[...] = jax.nn.relu(m / (db + 1e-16))


def _norm(acc):
    return pl.pallas_call(
        _norm_body,
        grid=(NPAD // _BLK,),
        in_specs=[pl.BlockSpec((NC, _BLK, WMSG), lambda i: (0, i, 0)),
                  pl.BlockSpec((16, HID), lambda i: (0, 0))],
        out_specs=pl.BlockSpec((_BLK, HID), lambda i: (i, 0)),
        out_shape=jax.ShapeDtypeStruct((NPAD, HID), jnp.float32),
    )(acc, jnp.asarray(_R_EXPAND))


def _edge_pass(p_ext, a_d, j_idx, i_idx, zacc):
    KE = 64
    R = 3
    n_chunks = E // KE
    base_ch = n_chunks // NW
    rem_ch = n_chunks % NW
    rows_t = NPAD // NS
    mesh = plsc.VectorSubcoreMesh(core_axis_name="c", subcore_axis_name="s")

    @functools.partial(
        pl.kernel,
        out_type=jax.ShapeDtypeStruct((NC, NPAD, WMSG), jnp.float32),
        mesh=mesh,
        compiler_params=pltpu.CompilerParams(use_tc_tiling_on_sc=False),
        scratch_types=[
            pltpu.VMEM((R, KE), jnp.int32),
            pltpu.VMEM((R, KE), jnp.int32),
            pltpu.VMEM((R, KE, WMSG), jnp.float32),
            pltpu.VMEM((R, KE, 16), jnp.float32),
            pltpu.VMEM_SHARED((NPAD, WMSG), jnp.float32),
            pltpu.SemaphoreType.DMA((R,)),
            pltpu.SemaphoreType.DMA((R,)),
            pltpu.SemaphoreType.DMA((R,)),
        ],
    )
    def k(p_hbm, ad_hbm, j_hbm, i_hbm, zacc_hbm,
          acc_out,
          idx_j, idx_i, xs_rows, ad_rows,
          acc, gx_sem, ga_sem, sc_sem):
        c = lax.axis_index("c")
        s = lax.axis_index("s")
        wid = s * NC + c
        r0 = s * rows_t
        pltpu.sync_copy(zacc_hbm.at[pl.ds(r0, rows_t)],
                        acc.at[pl.ds(r0, rows_t)])
        plsc.subcore_barrier()

        nt = base_ch + jnp.where(wid < rem_ch, 1, 0)

        def gather_descs(slot):
            return (
                pltpu.make_async_copy(p_hbm.at[idx_j.at[slot]],
                                      xs_rows.at[slot], gx_sem.at[slot]),
                pltpu.make_async_copy(ad_hbm.at[idx_i.at[slot]],
                                      ad_rows.at[slot], ga_sem.at[slot]),
            )

        def scatter_desc(slot):
            return pltpu.make_async_copy(xs_rows.at[slot],
                                         acc.at[idx_i.at[slot]],
                                         sc_sem.at[slot])

        def start(u):
            slot = u % R
            base = (wid + NW * u) * KE
            pltpu.sync_copy(j_hbm.at[pl.ds(base, KE)], idx_j.at[slot])
            pltpu.sync_copy(i_hbm.at[pl.ds(base, KE)], idx_i.at[slot])
            g0, g1 = gather_descs(slot)
            g0.start()
            g1.start()

        start(0)
        start(1)

        def chunk_body(t, carry):
            slot = t % R

            @pl.when(t + 2 < nt)
            def _prefetch():
                @pl.when(t >= 1)
                def _drain():
                    scatter_desc((t + 2) % R).wait()
                start(t + 2)

            g0, g1 = gather_descs(slot)
            g0.wait()
            g1.wait()

            def edge_body(e, inner):
                al = xs_rows[slot, e, pl.ds(HID, 16)] + ad_rows[slot, e]
                al = jnp.where(al >= 0.0, al, al * 0.2)
                ex = jnp.exp(al)
                xs_rows[slot, e, pl.ds(HID, 16)] = ex
                for h in range(H):
                    sl = pl.ds(h * DH, DH)
                    xs_rows[slot, e, sl] = ex[h] * xs_rows[slot, e, sl]
                return inner

            lax.fori_loop(0, KE, edge_body, 0)
            pltpu.async_copy(xs_rows.at[slot], acc.at[idx_i.at[slot]],
                             sc_sem.at[slot], add=True)
            return carry

        lax.fori_loop(0, nt, chunk_body, 0)
        scatter_desc((nt - 1) % R).wait()
        scatter_desc((nt - 2) % R).wait()
        scatter_desc(nt % R).wait()
        plsc.subcore_barrier()
        pltpu.sync_copy(acc.at[pl.ds(r0, rows_t)],
                        acc_out.at[c, pl.ds(r0, rows_t)])

    return k(p_ext, a_d, j_idx, i_idx, zacc)


def _pair_dot(t1a, t1b, t2a, t2b, m_idx, d_idx):
    n_chunks = ESP // K
    base_ch = n_chunks // NW
    rem_ch = n_chunks % NW
    mesh = plsc.VectorSubcoreMesh(core_axis_name="c", subcore_axis_name="s")

    @functools.partial(
        pl.kernel,
        out_type=jax.ShapeDtypeStruct((ESP,), jnp.float32),
        mesh=mesh,
        compiler_params=pltpu.CompilerParams(use_tc_tiling_on_sc=False),
        scratch_types=[
            pltpu.VMEM((K,), jnp.int32),
            pltpu.VMEM((K,), jnp.int32),
            pltpu.VMEM((K, HID), jnp.float32),
            pltpu.VMEM((K, HID), jnp.float32),
            pltpu.VMEM((K, HID), jnp.float32),
            pltpu.VMEM((K, HID), jnp.float32),
            pltpu.VMEM((K,), jnp.float32),
            pltpu.SemaphoreType.DMA,
            pltpu.SemaphoreType.DMA,
            pltpu.SemaphoreType.DMA,
            pltpu.SemaphoreType.DMA,
        ],
    )
    def k(t1a_hbm, t1b_hbm, t2a_hbm, t2b_hbm, m_hbm, d_hbm, y_hbm,
          mi, di, r1a, r1b, r2a, r2b, ybuf, sa, sb, sc2, sd):
        c = lax.axis_index("c")
        s = lax.axis_index("s")
        wid = s * NC + c
        nt = base_ch + jnp.where(wid < rem_ch, 1, 0)

        def chunk_body(t, carry):
            ch = wid + NW * t
            base = ch * K
            pltpu.sync_copy(m_hbm.at[pl.ds(base, K)], mi)
            pltpu.sync_copy(d_hbm.at[pl.ds(base, K)], di)
            cpa = pltpu.async_copy(t1a_hbm.at[mi], r1a, sa)
            cpb = pltpu.async_copy(t1b_hbm.at[mi], r1b, sb)
            cpc = pltpu.async_copy(t2a_hbm.at[di], r2a, sc2)
            cpd = pltpu.async_copy(t2b_hbm.at[di], r2b, sd)
            cpa.wait()
            cpb.wait()
            cpc.wait()
            cpd.wait()

            lanes = lax.iota(jnp.int32, 16)

            def group_body(g, carry):
                def pair_body(p, y16):
                    e = g * 16 + p
                    acc = r1a[e, pl.ds(0, 16)] * r2a[e, pl.ds(0, 16)]
                    acc = acc + r1b[e, pl.ds(0, 16)] * r2b[e, pl.ds(0, 16)]
                    for hh in range(1, HID // 16):
                        sl = pl.ds(hh * 16, 16)
                        acc = acc + r1a[e, sl] * r2a[e, sl]
                        acc = acc + r1b[e, sl] * r2b[e, sl]
                    for sh in (8, 4, 2, 1):
                        acc = acc + acc.at[lanes ^ sh].get(
                            mode="promise_in_bounds")
                    return jnp.where(lanes == p, acc, y16)

                y16 = lax.fori_loop(0, 16, pair_body,
                                    jnp.zeros((16,), jnp.float32))
                ybuf[pl.ds(g * 16, 16)] = y16
                return carry

            lax.fori_loop(0, K // 16, group_body, 0)
            pltpu.sync_copy(ybuf, y_hbm.at[pl.ds(base, K)])
            return carry

        lax.fori_loop(0, nt, chunk_body, 0)

    return k(t1a, t1b, t2a, t2b, m_idx, d_idx)


def _amat(att):
    eye = jnp.eye(H, dtype=jnp.float32)
    return (att[:, :, None] * eye[:, None, :]).reshape(HID, H)


def kernel(x_n1, x_n2, edge_index_n12, edge_index_n21, edge_index, params):
    f32 = jnp.float32
    x1 = jnp.pad(x_n1.astype(f32), ((0, NPAD - N), (0, 0)))
    x2 = jnp.pad(x_n2.astype(f32), ((0, NPAD - N), (0, 0)))
    j12 = edge_index_n12[0].astype(jnp.int32)
    i12 = edge_index_n12[1].astype(jnp.int32)
    j21 = edge_index_n21[0].astype(jnp.int32)
    i21 = edge_index_n21[1].astype(jnp.int32)
    mi = jnp.pad(edge_index[0].astype(jnp.int32), (0, ESP - ES))
    di = jnp.pad(edge_index[1].astype(jnp.int32), (0, ESP - ES))
    zacc = jnp.zeros((NPAD, WMSG), f32)

    h1 = _mm_relu(x1, params['lin']['n1']['W'], params['lin']['n1']['b'])
    h2 = _mm_relu(x2, params['lin']['n2']['W'], params['lin']['n2']['b'])

    pad8 = ((0, 0), (0, 16 - H))
    outs = []
    for lp in params['layers']:
        ams1 = jnp.pad(_amat(lp['att']['n1->n2']['src']), pad8)
        amd1 = jnp.pad(_amat(lp['att']['n2->n1']['dst']), pad8)
        ams2 = jnp.pad(_amat(lp['att']['n2->n1']['src']), pad8)
        amd2 = jnp.pad(_amat(lp['att']['n1->n2']['dst']), pad8)

        p1, a1d = _proj(h1, lp['proj']['n1']['W'], lp['proj']['n1']['b'],
                        ams1, amd1)
        p2, a2d = _proj(h2, lp['proj']['n2']['W'], lp['proj']['n2']['b'],
                        ams2, amd2)

        acc2 = _edge_pass(p1, a2d, j12, i12, zacc)
        acc1 = _edge_pass(p2, a1d, j21, i21, zacc)
        h1 = _norm(acc1)
        h2 = _norm(acc2)
        outs.append((h1, h2))

    y = _pair_dot(outs[0][0], outs[1][0], outs[0][1], outs[1][1], mi, di)
    return y[:ES].reshape(ES, 1)


# speedup vs baseline: 48.9550x; 1.0098x over previous
import functools

import jax
import jax.numpy as jnp
import numpy as np
from jax import lax
from jax.experimental import pallas as pl
from jax.experimental.pallas import tpu as pltpu
from jax.experimental.pallas import tpu_sc as plsc

N = 10000
NPAD = 10112
HID = 128
H = 8
DH = 16
E = 320000
ES = 100000
ESP = 100096

NC = 2
NS = 16
NW = NC * NS
K = 128
WMSG = HID + 16

_BLK = 632


def _mm_relu_body(x_ref, w_ref, b_ref, o_ref):
    o_ref[...] = jax.nn.relu(
        jnp.dot(x_ref[...], w_ref[...], preferred_element_type=jnp.float32)
        + b_ref[...])


def _mm_relu(x, w, b):
    return pl.pallas_call(
        _mm_relu_body,
        grid=(NPAD // _BLK,),
        in_specs=[pl.BlockSpec((_BLK, HID), lambda i: (i, 0)),
                  pl.BlockSpec((HID, HID), lambda i: (0, 0)),
                  pl.BlockSpec((1, HID), lambda i: (0, 0))],
        out_specs=pl.BlockSpec((_BLK, HID), lambda i: (i, 0)),
        out_shape=jax.ShapeDtypeStruct((NPAD, HID), jnp.float32),
    )(x, w, b.reshape(1, HID))


def _proj_body(x_ref, w_ref, b_ref, ams_ref, amd_ref, pe_ref, ad_ref):
    p = (jnp.dot(x_ref[...], w_ref[...], preferred_element_type=jnp.float32)
         + b_ref[...])
    pe_ref[:, :HID] = p
    pe_ref[:, HID:WMSG] = jnp.dot(p, ams_ref[...],
                                  preferred_element_type=jnp.float32)
    ad_ref[...] = jnp.dot(p, amd_ref[...], preferred_element_type=jnp.float32)


def _proj(x, w, b, ams, amd):
    return pl.pallas_call(
        _proj_body,
        grid=(NPAD // _BLK,),
        in_specs=[pl.BlockSpec((_BLK, HID), lambda i: (i, 0)),
                  pl.BlockSpec((HID, HID), lambda i: (0, 0)),
                  pl.BlockSpec((1, HID), lambda i: (0, 0)),
                  pl.BlockSpec((HID, 16), lambda i: (0, 0)),
                  pl.BlockSpec((HID, 16), lambda i: (0, 0))],
        out_specs=[pl.BlockSpec((_BLK, WMSG), lambda i: (i, 0)),
                   pl.BlockSpec((_BLK, 16), lambda i: (i, 0))],
        out_shape=[jax.ShapeDtypeStruct((NPAD, WMSG), jnp.float32),
                   jax.ShapeDtypeStruct((NPAD, 16), jnp.float32)],
    )(x, w, b.reshape(1, HID), ams, amd)


_R_EXPAND = np.zeros((16, HID), np.float32)
for _h in range(H):
    _R_EXPAND[_h, _h * DH:(_h + 1) * DH] = 1.0


def _norm_body(acc_ref, r_ref, o_ref):
    a = acc_ref[0] + acc_ref[1]
    m = a[:, :HID]
    d = a[:, HID:WMSG]
    db = jnp.dot(d, r_ref[...], preferred_element_type=jnp.float32)
    o_ref[...] = jax.nn.relu(m / (db + 1e-16))


def _norm(acc):
    return pl.pallas_call(
        _norm_body,
        grid=(NPAD // _BLK,),
        in_specs=[pl.BlockSpec((NC, _BLK, WMSG), lambda i: (0, i, 0)),
                  pl.BlockSpec((16, HID), lambda i: (0, 0))],
        out_specs=pl.BlockSpec((_BLK, HID), lambda i: (i, 0)),
        out_shape=jax.ShapeDtypeStruct((NPAD, HID), jnp.float32),
    )(acc, jnp.asarray(_R_EXPAND))


def _edge_pass(p_ext, a_d, j_idx, i_idx, zacc):
    KE = 64
    R = 3
    n_chunks = E // KE
    base_ch = n_chunks // NW
    rem_ch = n_chunks % NW
    rows_t = NPAD // NS
    mesh = plsc.VectorSubcoreMesh(core_axis_name="c", subcore_axis_name="s")

    @functools.partial(
        pl.kernel,
        out_type=jax.ShapeDtypeStruct((NC, NPAD, WMSG), jnp.float32),
        mesh=mesh,
        compiler_params=pltpu.CompilerParams(use_tc_tiling_on_sc=False),
        scratch_types=[
            pltpu.VMEM((R, KE), jnp.int32),
            pltpu.VMEM((R, KE), jnp.int32),
            pltpu.VMEM((R, KE, WMSG), jnp.float32),
            pltpu.VMEM((R, KE, 16), jnp.float32),
            pltpu.VMEM_SHARED((NPAD, WMSG), jnp.float32),
            pltpu.SemaphoreType.DMA((R,)),
            pltpu.SemaphoreType.DMA((R,)),
            pltpu.SemaphoreType.DMA((R,)),
        ],
    )
    def k(p_hbm, ad_hbm, j_hbm, i_hbm, zacc_hbm,
          acc_out,
          idx_j, idx_i, xs_rows, ad_rows,
          acc, gx_sem, ga_sem, sc_sem):
        c = lax.axis_index("c")
        s = lax.axis_index("s")
        wid = s * NC + c
        r0 = s * rows_t
        pltpu.sync_copy(zacc_hbm.at[pl.ds(r0, rows_t)],
                        acc.at[pl.ds(r0, rows_t)])
        plsc.subcore_barrier()

        nt = base_ch + jnp.where(wid < rem_ch, 1, 0)

        def gather_descs(slot):
            return (
                pltpu.make_async_copy(p_hbm.at[idx_j.at[slot]],
                                      xs_rows.at[slot], gx_sem.at[slot]),
                pltpu.make_async_copy(ad_hbm.at[idx_i.at[slot]],
                                      ad_rows.at[slot], ga_sem.at[slot]),
            )

        def scatter_desc(slot):
            return pltpu.make_async_copy(xs_rows.at[slot],
                                         acc.at[idx_i.at[slot]],
                                         sc_sem.at[slot])

        def start(u):
            slot = u % R
            base = (wid + NW * u) * KE
            pltpu.sync_copy(j_hbm.at[pl.ds(base, KE)], idx_j.at[slot])
            pltpu.sync_copy(i_hbm.at[pl.ds(base, KE)], idx_i.at[slot])
            g0, g1 = gather_descs(slot)
            g0.start()
            g1.start()

        start(0)
        start(1)

        def chunk_body(t, carry):
            slot = t % R

            @pl.when(t + 2 < nt)
            def _prefetch():
                @pl.when(t >= 1)
                def _drain():
                    scatter_desc((t + 2) % R).wait()
                start(t + 2)

            g0, g1 = gather_descs(slot)
            g0.wait()
            g1.wait()

            def edge_body(e, inner):
                al = xs_rows[slot, e, pl.ds(HID, 16)] + ad_rows[slot, e]
                al = jnp.where(al >= 0.0, al, al * 0.2)
                ex = jnp.exp(al)
                xs_rows[slot, e, pl.ds(HID, 16)] = ex
                for h in range(H):
                    sl = pl.ds(h * DH, DH)
                    exh = ex.at[jnp.full((16,), h, jnp.int32)].get(
                        mode="promise_in_bounds")
                    xs_rows[slot, e, sl] = exh * xs_rows[slot, e, sl]
                return inner

            lax.fori_loop(0, KE, edge_body, 0, unroll=2)
            pltpu.async_copy(xs_rows.at[slot], acc.at[idx_i.at[slot]],
                             sc_sem.at[slot], add=True)
            return carry

        lax.fori_loop(0, nt, chunk_body, 0)
        scatter_desc((nt - 1) % R).wait()
        scatter_desc((nt - 2) % R).wait()
        scatter_desc(nt % R).wait()
        plsc.subcore_barrier()
        pltpu.sync_copy(acc.at[pl.ds(r0, rows_t)],
                        acc_out.at[c, pl.ds(r0, rows_t)])

    return k(p_ext, a_d, j_idx, i_idx, zacc)


def _pair_dot(t1a, t1b, t2a, t2b, m_idx, d_idx):
    n_chunks = ESP // K
    base_ch = n_chunks // NW
    rem_ch = n_chunks % NW
    mesh = plsc.VectorSubcoreMesh(core_axis_name="c", subcore_axis_name="s")

    @functools.partial(
        pl.kernel,
        out_type=jax.ShapeDtypeStruct((ESP,), jnp.float32),
        mesh=mesh,
        compiler_params=pltpu.CompilerParams(use_tc_tiling_on_sc=False),
        scratch_types=[
            pltpu.VMEM((K,), jnp.int32),
            pltpu.VMEM((K,), jnp.int32),
            pltpu.VMEM((K, HID), jnp.float32),
            pltpu.VMEM((K, HID), jnp.float32),
            pltpu.VMEM((K, HID), jnp.float32),
            pltpu.VMEM((K, HID), jnp.float32),
            pltpu.VMEM((K,), jnp.float32),
            pltpu.SemaphoreType.DMA,
            pltpu.SemaphoreType.DMA,
            pltpu.SemaphoreType.DMA,
            pltpu.SemaphoreType.DMA,
        ],
    )
    def k(t1a_hbm, t1b_hbm, t2a_hbm, t2b_hbm, m_hbm, d_hbm, y_hbm,
          mi, di, r1a, r1b, r2a, r2b, ybuf, sa, sb, sc2, sd):
        c = lax.axis_index("c")
        s = lax.axis_index("s")
        wid = s * NC + c
        nt = base_ch + jnp.where(wid < rem_ch, 1, 0)

        def chunk_body(t, carry):
            ch = wid + NW * t
            base = ch * K
            pltpu.sync_copy(m_hbm.at[pl.ds(base, K)], mi)
            pltpu.sync_copy(d_hbm.at[pl.ds(base, K)], di)
            cpa = pltpu.async_copy(t1a_hbm.at[mi], r1a, sa)
            cpb = pltpu.async_copy(t1b_hbm.at[mi], r1b, sb)
            cpc = pltpu.async_copy(t2a_hbm.at[di], r2a, sc2)
            cpd = pltpu.async_copy(t2b_hbm.at[di], r2b, sd)
            cpa.wait()
            cpb.wait()
            cpc.wait()
            cpd.wait()

            lanes = lax.iota(jnp.int32, 16)

            def group_body(g, carry):
                def pair_body(p, y16):
                    e = g * 16 + p
                    acc = r1a[e, pl.ds(0, 16)] * r2a[e, pl.ds(0, 16)]
                    acc = acc + r1b[e, pl.ds(0, 16)] * r2b[e, pl.ds(0, 16)]
                    for hh in range(1, HID // 16):
                        sl = pl.ds(hh * 16, 16)
                        acc = acc + r1a[e, sl] * r2a[e, sl]
                        acc = acc + r1b[e, sl] * r2b[e, sl]
                    for sh in (8, 4, 2, 1):
                        acc = acc + acc.at[lanes ^ sh].get(
                            mode="promise_in_bounds")
                    return jnp.where(lanes == p, acc, y16)

                y16 = lax.fori_loop(0, 16, pair_body,
                                    jnp.zeros((16,), jnp.float32))
                ybuf[pl.ds(g * 16, 16)] = y16
                return carry

            lax.fori_loop(0, K // 16, group_body, 0)
            pltpu.sync_copy(ybuf, y_hbm.at[pl.ds(base, K)])
            return carry

        lax.fori_loop(0, nt, chunk_body, 0)

    return k(t1a, t1b, t2a, t2b, m_idx, d_idx)


def _amat(att):
    eye = jnp.eye(H, dtype=jnp.float32)
    return (att[:, :, None] * eye[:, None, :]).reshape(HID, H)


def kernel(x_n1, x_n2, edge_index_n12, edge_index_n21, edge_index, params):
    f32 = jnp.float32
    x1 = jnp.pad(x_n1.astype(f32), ((0, NPAD - N), (0, 0)))
    x2 = jnp.pad(x_n2.astype(f32), ((0, NPAD - N), (0, 0)))
    j12 = edge_index_n12[0].astype(jnp.int32)
    i12 = edge_index_n12[1].astype(jnp.int32)
    j21 = edge_index_n21[0].astype(jnp.int32)
    i21 = edge_index_n21[1].astype(jnp.int32)
    mi = jnp.pad(edge_index[0].astype(jnp.int32), (0, ESP - ES))
    di = jnp.pad(edge_index[1].astype(jnp.int32), (0, ESP - ES))
    zacc = jnp.zeros((NPAD, WMSG), f32)

    h1 = _mm_relu(x1, params['lin']['n1']['W'], params['lin']['n1']['b'])
    h2 = _mm_relu(x2, params['lin']['n2']['W'], params['lin']['n2']['b'])

    pad8 = ((0, 0), (0, 16 - H))
    outs = []
    for lp in params['layers']:
        ams1 = jnp.pad(_amat(lp['att']['n1->n2']['src']), pad8)
        amd1 = jnp.pad(_amat(lp['att']['n2->n1']['dst']), pad8)
        ams2 = jnp.pad(_amat(lp['att']['n2->n1']['src']), pad8)
        amd2 = jnp.pad(_amat(lp['att']['n1->n2']['dst']), pad8)

        p1, a1d = _proj(h1, lp['proj']['n1']['W'], lp['proj']['n1']['b'],
                        ams1, amd1)
        p2, a2d = _proj(h2, lp['proj']['n2']['W'], lp['proj']['n2']['b'],
                        ams2, amd2)

        acc2 = _edge_pass(p1, a2d, j12, i12, zacc)
        acc1 = _edge_pass(p2, a1d, j21, i21, zacc)
        h1 = _norm(acc1)
        h2 = _norm(acc2)
        outs.append((h1, h2))

    y = _pair_dot(outs[0][0], outs[1][0], outs[0][1], outs[1][1], mi, di)
    return y[:ES].reshape(ES, 1)


# speedup vs baseline: 101.9390x; 2.0823x over previous
import functools

import jax
import jax.numpy as jnp
import numpy as np
from jax import lax
from jax.experimental import pallas as pl
from jax.experimental.pallas import tpu as pltpu
from jax.experimental.pallas import tpu_sc as plsc

N = 10000
NPAD = 10112
HID = 128
H = 8
DH = 16
E = 320000
ES = 100000
ESP = 100096

NC = 2
NS = 16
NW = NC * NS
K = 128
WMSG = HID + 16

_BLK = 632


def _mm_relu_body(x_ref, w_ref, b_ref, o_ref):
    o_ref[...] = jax.nn.relu(
        jnp.dot(x_ref[...], w_ref[...], preferred_element_type=jnp.float32)
        + b_ref[...])


def _mm_relu(x, w, b):
    return pl.pallas_call(
        _mm_relu_body,
        grid=(NPAD // _BLK,),
        in_specs=[pl.BlockSpec((_BLK, HID), lambda i: (i, 0)),
                  pl.BlockSpec((HID, HID), lambda i: (0, 0)),
                  pl.BlockSpec((1, HID), lambda i: (0, 0))],
        out_specs=pl.BlockSpec((_BLK, HID), lambda i: (i, 0)),
        out_shape=jax.ShapeDtypeStruct((NPAD, HID), jnp.float32),
    )(x, w, b.reshape(1, HID))


def _proj_body(x_ref, w_ref, b_ref, ams_ref, amd_ref, pe_ref, ad_ref):
    p = (jnp.dot(x_ref[...], w_ref[...], preferred_element_type=jnp.float32)
         + b_ref[...])
    pe_ref[:, :HID] = p
    pe_ref[:, HID:WMSG] = jnp.dot(p, ams_ref[...],
                                  preferred_element_type=jnp.float32)
    ad_ref[...] = jnp.dot(p, amd_ref[...], preferred_element_type=jnp.float32)


def _proj(x, w, b, ams, amd):
    return pl.pallas_call(
        _proj_body,
        grid=(NPAD // _BLK,),
        in_specs=[pl.BlockSpec((_BLK, HID), lambda i: (i, 0)),
                  pl.BlockSpec((HID, HID), lambda i: (0, 0)),
                  pl.BlockSpec((1, HID), lambda i: (0, 0)),
                  pl.BlockSpec((HID, 16), lambda i: (0, 0)),
                  pl.BlockSpec((HID, 16), lambda i: (0, 0))],
        out_specs=[pl.BlockSpec((_BLK, WMSG), lambda i: (i, 0)),
                   pl.BlockSpec((_BLK, 16), lambda i: (i, 0))],
        out_shape=[jax.ShapeDtypeStruct((NPAD, WMSG), jnp.float32),
                   jax.ShapeDtypeStruct((NPAD, 16), jnp.float32)],
    )(x, w, b.reshape(1, HID), ams, amd)


_R_EXPAND = np.zeros((16, HID), np.float32)
for _h in range(H):
    _R_EXPAND[_h, _h * DH:(_h + 1) * DH] = 1.0


def _norm_body(acc_ref, r_ref, o_ref):
    a = acc_ref[0] + acc_ref[1]
    m = a[:, :HID]
    d = a[:, HID:WMSG]
    db = jnp.dot(d, r_ref[...], preferred_element_type=jnp.float32)
    o_ref[...] = jax.nn.relu(m / (db + 1e-16))


def _norm(acc):
    return pl.pallas_call(
        _norm_body,
        grid=(NPAD // _BLK,),
        in_specs=[pl.BlockSpec((NC, _BLK, WMSG), lambda i: (0, i, 0)),
                  pl.BlockSpec((16, HID), lambda i: (0, 0))],
        out_specs=pl.BlockSpec((_BLK, HID), lambda i: (i, 0)),
        out_shape=jax.ShapeDtypeStruct((NPAD, HID), jnp.float32),
    )(acc, jnp.asarray(_R_EXPAND))


EPAD = 322560
KE = 40
NTC = EPAD // NW // KE


def _edge_pass(p_ext, a_d, j2d, i2d, zacc):
    rows_t = NPAD // NS
    mesh = plsc.VectorSubcoreMesh(core_axis_name="c", subcore_axis_name="s")

    @functools.partial(
        pl.kernel,
        out_type=jax.ShapeDtypeStruct((NC, NPAD, WMSG), jnp.float32),
        mesh=mesh,
        compiler_params=pltpu.CompilerParams(use_tc_tiling_on_sc=False),
        scratch_types=[
            pltpu.VMEM((NTC, KE), jnp.int32),
            pltpu.VMEM((NTC, KE), jnp.int32),
            pltpu.VMEM((3, KE, WMSG), jnp.float32),
            pltpu.VMEM((3, KE, 16), jnp.float32),
            pltpu.VMEM_SHARED((NPAD, WMSG), jnp.float32),
            pltpu.SemaphoreType.DMA((3,)),
            pltpu.SemaphoreType.DMA((3,)),
            pltpu.SemaphoreType.DMA((3,)),
        ],
    )
    def k(p_hbm, ad_hbm, j_hbm, i_hbm, zacc_hbm,
          acc_out,
          jrows, irows, xs_rows, ad_rows,
          acc, gx_sem, ga_sem, sc_sem):
        c = lax.axis_index("c")
        s = lax.axis_index("s")
        wid = s * NC + c
        r0 = s * rows_t
        pltpu.sync_copy(j_hbm.at[pl.ds(wid * NTC, NTC)], jrows)
        pltpu.sync_copy(i_hbm.at[pl.ds(wid * NTC, NTC)], irows)
        pltpu.sync_copy(zacc_hbm.at[pl.ds(r0, rows_t)],
                        acc.at[pl.ds(r0, rows_t)])
        plsc.subcore_barrier()

        def gather_descs(ch, slot):
            return (
                pltpu.make_async_copy(p_hbm.at[jrows.at[ch]],
                                      xs_rows.at[slot], gx_sem.at[slot]),
                pltpu.make_async_copy(ad_hbm.at[irows.at[ch]],
                                      ad_rows.at[slot], ga_sem.at[slot]),
            )

        def scatter_desc(ch, slot):
            return pltpu.make_async_copy(xs_rows.at[slot],
                                         acc.at[irows.at[ch]],
                                         sc_sem.at[slot])

        def start(ch, slot):
            g0, g1 = gather_descs(ch, slot)
            g0.start()
            g1.start()

        start(0, 0)

        def step(ch, slot):
            @pl.when(ch + 1 < NTC)
            def _prefetch():
                @pl.when(ch >= 2)
                def _drain():
                    scatter_desc(ch - 2, (slot + 1) % 3).wait()
                start(ch + 1, (slot + 1) % 3)

            g0, g1 = gather_descs(ch, slot)
            g0.wait()
            g1.wait()

            for e in range(KE):
                al = xs_rows[slot, e, pl.ds(HID, 16)] + ad_rows[slot, e]
                al = jnp.where(al >= 0.0, al, al * 0.2)
                ex = jnp.exp(al)
                xs_rows[slot, e, pl.ds(HID, 16)] = ex
                for h in range(H):
                    sl = pl.ds(h * DH, DH)
                    exh = ex.at[jnp.full((16,), h, jnp.int32)].get(
                        mode="promise_in_bounds")
                    xs_rows[slot, e, sl] = exh * xs_rows[slot, e, sl]

            pltpu.async_copy(xs_rows.at[slot], acc.at[irows.at[ch]],
                             sc_sem.at[slot], add=True)

        def outer_body(u, carry):
            step(3 * u, 0)
            step(3 * u + 1, 1)
            step(3 * u + 2, 2)
            return carry

        lax.fori_loop(0, NTC // 3, outer_body, 0)
        scatter_desc(NTC - 3, 0).wait()
        scatter_desc(NTC - 2, 1).wait()
        scatter_desc(NTC - 1, 2).wait()
        plsc.subcore_barrier()
        pltpu.sync_copy(acc.at[pl.ds(r0, rows_t)],
                        acc_out.at[c, pl.ds(r0, rows_t)])

    return k(p_ext, a_d, j2d, i2d, zacc)


def _pair_dot(t1a, t1b, t2a, t2b, m_idx, d_idx):
    n_chunks = ESP // K
    base_ch = n_chunks // NW
    rem_ch = n_chunks % NW
    mesh = plsc.VectorSubcoreMesh(core_axis_name="c", subcore_axis_name="s")

    @functools.partial(
        pl.kernel,
        out_type=jax.ShapeDtypeStruct((ESP,), jnp.float32),
        mesh=mesh,
        compiler_params=pltpu.CompilerParams(use_tc_tiling_on_sc=False),
        scratch_types=[
            pltpu.VMEM((K,), jnp.int32),
            pltpu.VMEM((K,), jnp.int32),
            pltpu.VMEM((K, HID), jnp.float32),
            pltpu.VMEM((K, HID), jnp.float32),
            pltpu.VMEM((K, HID), jnp.float32),
            pltpu.VMEM((K, HID), jnp.float32),
            pltpu.VMEM((K,), jnp.float32),
            pltpu.SemaphoreType.DMA,
            pltpu.SemaphoreType.DMA,
            pltpu.SemaphoreType.DMA,
            pltpu.SemaphoreType.DMA,
        ],
    )
    def k(t1a_hbm, t1b_hbm, t2a_hbm, t2b_hbm, m_hbm, d_hbm, y_hbm,
          mi, di, r1a, r1b, r2a, r2b, ybuf, sa, sb, sc2, sd):
        c = lax.axis_index("c")
        s = lax.axis_index("s")
        wid = s * NC + c
        nt = base_ch + jnp.where(wid < rem_ch, 1, 0)

        def chunk_body(t, carry):
            ch = wid + NW * t
            base = ch * K
            pltpu.sync_copy(m_hbm.at[pl.ds(base, K)], mi)
            pltpu.sync_copy(d_hbm.at[pl.ds(base, K)], di)
            cpa = pltpu.async_copy(t1a_hbm.at[mi], r1a, sa)
            cpb = pltpu.async_copy(t1b_hbm.at[mi], r1b, sb)
            cpc = pltpu.async_copy(t2a_hbm.at[di], r2a, sc2)
            cpd = pltpu.async_copy(t2b_hbm.at[di], r2b, sd)
            cpa.wait()
            cpb.wait()
            cpc.wait()
            cpd.wait()

            lanes = lax.iota(jnp.int32, 16)

            def group_body(g, carry):
                def pair_body(p, y16):
                    e = g * 16 + p
                    acc = r1a[e, pl.ds(0, 16)] * r2a[e, pl.ds(0, 16)]
                    acc = acc + r1b[e, pl.ds(0, 16)] * r2b[e, pl.ds(0, 16)]
                    for hh in range(1, HID // 16):
                        sl = pl.ds(hh * 16, 16)
                        acc = acc + r1a[e, sl] * r2a[e, sl]
                        acc = acc + r1b[e, sl] * r2b[e, sl]
                    for sh in (8, 4, 2, 1):
                        acc = acc + acc.at[lanes ^ sh].get(
                            mode="promise_in_bounds")
                    return jnp.where(lanes == p, acc, y16)

                y16 = lax.fori_loop(0, 16, pair_body,
                                    jnp.zeros((16,), jnp.float32))
                ybuf[pl.ds(g * 16, 16)] = y16
                return carry

            lax.fori_loop(0, K // 16, group_body, 0)
            pltpu.sync_copy(ybuf, y_hbm.at[pl.ds(base, K)])
            return carry

        lax.fori_loop(0, nt, chunk_body, 0)

    return k(t1a, t1b, t2a, t2b, m_idx, d_idx)


def _amat(att):
    eye = jnp.eye(H, dtype=jnp.float32)
    return (att[:, :, None] * eye[:, None, :]).reshape(HID, H)


def kernel(x_n1, x_n2, edge_index_n12, edge_index_n21, edge_index, params):
    f32 = jnp.float32
    x1 = jnp.pad(x_n1.astype(f32), ((0, NPAD - N), (0, 0)))
    x2 = jnp.pad(x_n2.astype(f32), ((0, NPAD - N), (0, 0)))
    def _epad(ei):
        j = jnp.pad(ei[0].astype(jnp.int32), (0, EPAD - E))
        i = jnp.pad(ei[1].astype(jnp.int32), (0, EPAD - E),
                    constant_values=N)
        return j.reshape(EPAD // KE, KE), i.reshape(EPAD // KE, KE)

    j12, i12 = _epad(edge_index_n12)
    j21, i21 = _epad(edge_index_n21)
    mi = jnp.pad(edge_index[0].astype(jnp.int32), (0, ESP - ES))
    di = jnp.pad(edge_index[1].astype(jnp.int32), (0, ESP - ES))
    zacc = jnp.zeros((NPAD, WMSG), f32)

    h1 = _mm_relu(x1, params['lin']['n1']['W'], params['lin']['n1']['b'])
    h2 = _mm_relu(x2, params['lin']['n2']['W'], params['lin']['n2']['b'])

    pad8 = ((0, 0), (0, 16 - H))
    outs = []
    for lp in params['layers']:
        ams1 = jnp.pad(_amat(lp['att']['n1->n2']['src']), pad8)
        amd1 = jnp.pad(_amat(lp['att']['n2->n1']['dst']), pad8)
        ams2 = jnp.pad(_amat(lp['att']['n2->n1']['src']), pad8)
        amd2 = jnp.pad(_amat(lp['att']['n1->n2']['dst']), pad8)

        p1, a1d = _proj(h1, lp['proj']['n1']['W'], lp['proj']['n1']['b'],
                        ams1, amd1)
        p2, a2d = _proj(h2, lp['proj']['n2']['W'], lp['proj']['n2']['b'],
                        ams2, amd2)

        acc2 = _edge_pass(p1, a2d, j12, i12, zacc)
        acc1 = _edge_pass(p2, a1d, j21, i21, zacc)
        h1 = _norm(acc1)
        h2 = _norm(acc2)
        outs.append((h1, h2))

    y = _pair_dot(outs[0][0], outs[1][0], outs[0][1], outs[1][1], mi, di)
    return y[:ES].reshape(ES, 1)
